# baseline (speedup 1.0000x reference)
import jax
import jax.numpy as jnp
from jax.experimental import pallas as pl
from jax.experimental.pallas import tpu as pltpu

_ROWS = 2048


def _row_argmax_t(z):
    n = z.shape[1]
    zmax = jnp.max(z, axis=1, keepdims=True)
    ii = jax.lax.broadcasted_iota(jnp.int32, z.shape, 1)
    k = jnp.min(jnp.where(z >= zmax, ii, n), axis=1, keepdims=True)
    return k.T


def _count_ge16(v16, trial):
    v = jnp.where(v16 >= trial, jnp.int16(1), jnp.int16(0))
    while v.shape[0] > 1:
        half = v.shape[0] // 2
        v = v[:half] + v[half:]
    return v


def _search16(keys, need):
    thr = jnp.where(_count_ge16(keys, jnp.int16(0)) >= need,
                    jnp.int16(0), jnp.int16(-32768))
    for bit in range(14, -1, -1):
        trial = thr | jnp.int16(1 << bit)
        thr = jnp.where(_count_ge16(keys, trial) >= need, trial, thr)
    return thr


def _kwta(x, kt):
    b = jax.lax.bitcast_convert_type(x, jnp.int32)
    m = b ^ (jax.lax.shift_right_arithmetic(b, 31) & jnp.int32(0x7FFFFFFF))
    mt = m.T
    hi = (mt >> 16).astype(jnp.int16)
    lo = (((mt ^ 0x8000) << 16) >> 16).astype(jnp.int16)
    kt16 = kt.astype(jnp.int16)
    hiT = _search16(hi, kt16)
    cnt_gt = _count_ge16(jnp.where(hi > hiT, jnp.int16(0), jnp.int16(-32768)),
                         jnp.int16(0))
    mlo = jnp.where(hi == hiT, lo, jnp.int16(-32768))
    loT = _search16(mlo, kt16 - cnt_gt)
    thr = (hiT.astype(jnp.int32) << 16) + (loT.astype(jnp.int32) + 32768)
    scale_t = jnp.where(mt >= thr, 1.0, jnp.float32(1.0 / 3.0))
    return x * scale_t.T


def _body(ci_ref, wc11_ref, bc11_ref, wc12_ref, bc12_ref,
          wc21_ref, bc21_ref, wc22_ref, bc22_ref,
          wc31_ref, bc31_ref, wc32_ref, bc32_ref,
          w1_ref, b1_ref, w2_ref, b2_ref, w3_ref, b3_ref,
          w4_ref, b4_ref, out_ref):
    def dot(a, b):
        return jax.lax.dot_general(a, b, (((1,), (0,)), ((), ())),
                                   preferred_element_type=jnp.float32)

    ci = ci_ref[...]
    k1 = _row_argmax_t(dot(jnp.tanh(dot(ci, wc11_ref[...]) + bc11_ref[...]),
                           wc12_ref[...]) + bc12_ref[...])
    k2 = _row_argmax_t(dot(jnp.tanh(dot(ci, wc21_ref[...]) + bc21_ref[...]),
                           wc22_ref[...]) + bc22_ref[...])
    k3 = _row_argmax_t(dot(jnp.tanh(dot(ci, wc31_ref[...]) + bc31_ref[...]),
                           wc32_ref[...]) + bc32_ref[...])
    x = _kwta(dot(ci, w1_ref[...]) + b1_ref[...], k1)
    x = _kwta(dot(x, w2_ref[...]) + b2_ref[...], k2)
    x = _kwta(dot(x, w3_ref[...]) + b3_ref[...], k3)
    out_ref[...] = dot(x, w4_ref[...]) + b4_ref[...]


def kernel(state, action, task_indicator,
           w_cx1_1, b_cx1_1, w_cx1_2, b_cx1_2,
           w_cx2_1, b_cx2_1, w_cx2_2, b_cx2_2,
           w_cx3_1, b_cx3_1, w_cx3_2, b_cx3_2,
           w1, b1, w2, b2, w3, b3, w4, b4):
    b = state.shape[0]
    rows = min(_ROWS, b)
    ci = jnp.concatenate([state, task_indicator, action], axis=1)
    inp = ci.shape[1]
    h = w4.shape[1]

    weights = [w_cx1_1, b_cx1_1.reshape(1, -1), w_cx1_2, b_cx1_2.reshape(1, -1),
               w_cx2_1, b_cx2_1.reshape(1, -1), w_cx2_2, b_cx2_2.reshape(1, -1),
               w_cx3_1, b_cx3_1.reshape(1, -1), w_cx3_2, b_cx3_2.reshape(1, -1),
               w1, b1.reshape(1, -1), w2, b2.reshape(1, -1),
               w3, b3.reshape(1, -1), w4, b4.reshape(1, -1)]

    return pl.pallas_call(
        _body,
        grid=(b // rows,),
        in_specs=[pl.BlockSpec((rows, inp), lambda i: (i, 0))] +
                 [pl.BlockSpec(w.shape, lambda i: (0, 0)) for w in weights],
        out_specs=pl.BlockSpec((rows, h), lambda i: (i, 0)),
        out_shape=jax.ShapeDtypeStruct((b, h), jnp.float32),
    )(ci, *weights)


# speedup vs baseline: 1.2503x; 1.2503x over previous
import jax
import jax.numpy as jnp
from jax.experimental import pallas as pl
from jax.experimental.pallas import tpu as pltpu

_ROWS = 1024


def _row_argmax_t(z):
    n = z.shape[1]
    zmax = jnp.max(z, axis=1, keepdims=True)
    ii = jax.lax.broadcasted_iota(jnp.int32, z.shape, 1)
    k = jnp.min(jnp.where(z >= zmax, ii, n), axis=1, keepdims=True)
    return k.T


def _count_ge16(v16, trial):
    v = jnp.where(v16 >= trial, jnp.int16(1), jnp.int16(0))
    while v.shape[0] > 1:
        half = v.shape[0] // 2
        v = v[:half] + v[half:]
    return v


def _search16(keys, need):
    thr = jnp.where(_count_ge16(keys, jnp.int16(0)) >= need,
                    jnp.int16(0), jnp.int16(-32768))
    for bit in range(14, -1, -1):
        trial = thr | jnp.int16(1 << bit)
        thr = jnp.where(_count_ge16(keys, trial) >= need, trial, thr)
    return thr


def _kwta(x, kt):
    b = jax.lax.bitcast_convert_type(x, jnp.int32)
    m = b ^ (jax.lax.shift_right_arithmetic(b, 31) & jnp.int32(0x7FFFFFFF))
    mt = m.T
    hi = (mt >> 16).astype(jnp.int16)
    lo = mt.astype(jnp.int16) ^ jnp.int16(-32768)
    kt16 = kt.astype(jnp.int16)
    hiT = _search16(hi, kt16)
    cnt_gt = _count_ge16(jnp.where(hi > hiT, jnp.int16(0), jnp.int16(-32768)),
                         jnp.int16(0))
    mlo = jnp.where(hi == hiT, lo, jnp.int16(-32768))
    loT = _search16(mlo, kt16 - cnt_gt)
    thr = (hiT.astype(jnp.int32) << 16) + (loT.astype(jnp.int32) + 32768)
    scale_t = jnp.where(mt >= thr, 1.0, jnp.float32(1.0 / 3.0))
    return x * scale_t.T


def _body(ci_ref, wc11_ref, bc11_ref, wc12_ref, bc12_ref,
          wc21_ref, bc21_ref, wc22_ref, bc22_ref,
          wc31_ref, bc31_ref, wc32_ref, bc32_ref,
          w1_ref, b1_ref, w2_ref, b2_ref, w3_ref, b3_ref,
          w4_ref, b4_ref, out_ref):
    def dot(a, b):
        return jax.lax.dot_general(a, b, (((1,), (0,)), ((), ())),
                                   preferred_element_type=jnp.float32)

    ci = ci_ref[...]
    k1 = _row_argmax_t(dot(jnp.tanh(dot(ci, wc11_ref[...]) + bc11_ref[...]),
                           wc12_ref[...]) + bc12_ref[...])
    k2 = _row_argmax_t(dot(jnp.tanh(dot(ci, wc21_ref[...]) + bc21_ref[...]),
                           wc22_ref[...]) + bc22_ref[...])
    k3 = _row_argmax_t(dot(jnp.tanh(dot(ci, wc31_ref[...]) + bc31_ref[...]),
                           wc32_ref[...]) + bc32_ref[...])
    x = _kwta(dot(ci, w1_ref[...]) + b1_ref[...], k1)
    x = _kwta(dot(x, w2_ref[...]) + b2_ref[...], k2)
    x = _kwta(dot(x, w3_ref[...]) + b3_ref[...], k3)
    out_ref[...] = dot(x, w4_ref[...]) + b4_ref[...]


def kernel(state, action, task_indicator,
           w_cx1_1, b_cx1_1, w_cx1_2, b_cx1_2,
           w_cx2_1, b_cx2_1, w_cx2_2, b_cx2_2,
           w_cx3_1, b_cx3_1, w_cx3_2, b_cx3_2,
           w1, b1, w2, b2, w3, b3, w4, b4):
    b = state.shape[0]
    rows = min(_ROWS, b)
    ci = jnp.concatenate([state, task_indicator, action], axis=1)
    inp = ci.shape[1]
    h = w4.shape[1]

    weights = [w_cx1_1, b_cx1_1.reshape(1, -1), w_cx1_2, b_cx1_2.reshape(1, -1),
               w_cx2_1, b_cx2_1.reshape(1, -1), w_cx2_2, b_cx2_2.reshape(1, -1),
               w_cx3_1, b_cx3_1.reshape(1, -1), w_cx3_2, b_cx3_2.reshape(1, -1),
               w1, b1.reshape(1, -1), w2, b2.reshape(1, -1),
               w3, b3.reshape(1, -1), w4, b4.reshape(1, -1)]

    return pl.pallas_call(
        _body,
        grid=(b // rows,),
        in_specs=[pl.BlockSpec((rows, inp), lambda i: (i, 0))] +
                 [pl.BlockSpec(w.shape, lambda i: (0, 0)) for w in weights],
        out_specs=pl.BlockSpec((rows, h), lambda i: (i, 0)),
        out_shape=jax.ShapeDtypeStruct((b, h), jnp.float32),
    )(ci, *weights)


# speedup vs baseline: 1.2515x; 1.0009x over previous
import jax
import jax.numpy as jnp
from jax.experimental import pallas as pl
from jax.experimental.pallas import tpu as pltpu

_ROWS = 1024


def _row_argmax_t(z):
    n = z.shape[1]
    zmax = jnp.max(z, axis=1, keepdims=True)
    ii = jax.lax.broadcasted_iota(jnp.int32, z.shape, 1)
    k = jnp.min(jnp.where(z >= zmax, ii, n), axis=1, keepdims=True)
    return k.T


def _count_ge16(v16, trial):
    v = jnp.where(v16 >= trial, jnp.int16(1), jnp.int16(0))
    while v.shape[0] > 16:
        half = v.shape[0] // 2
        v = v[:half] + v[half:]
    return jnp.sum(v.astype(jnp.int32), axis=0,
                   keepdims=True).astype(jnp.int16)


def _search16(keys, need):
    thr = jnp.where(_count_ge16(keys, jnp.int16(0)) >= need,
                    jnp.int16(0), jnp.int16(-32768))
    for bit in range(14, -1, -1):
        trial = thr | jnp.int16(1 << bit)
        thr = jnp.where(_count_ge16(keys, trial) >= need, trial, thr)
    return thr


def _kwta(x, kt):
    b = jax.lax.bitcast_convert_type(x, jnp.int32)
    m = b ^ (jax.lax.shift_right_arithmetic(b, 31) & jnp.int32(0x7FFFFFFF))
    mt = m.T
    hi = (mt >> 16).astype(jnp.int16)
    lo = mt.astype(jnp.int16) ^ jnp.int16(-32768)
    kt16 = kt.astype(jnp.int16)
    hiT = _search16(hi, kt16)
    cnt_gt = _count_ge16(jnp.where(hi > hiT, jnp.int16(0), jnp.int16(-32768)),
                         jnp.int16(0))
    mlo = jnp.where(hi == hiT, lo, jnp.int16(-32768))
    loT = _search16(mlo, kt16 - cnt_gt)
    thr = (hiT.astype(jnp.int32) << 16) + (loT.astype(jnp.int32) + 32768)
    scale_t = jnp.where(mt >= thr, 1.0, jnp.float32(1.0 / 3.0))
    return x * scale_t.T


def _body(ci_ref, wc11_ref, bc11_ref, wc12_ref, bc12_ref,
          wc21_ref, bc21_ref, wc22_ref, bc22_ref,
          wc31_ref, bc31_ref, wc32_ref, bc32_ref,
          w1_ref, b1_ref, w2_ref, b2_ref, w3_ref, b3_ref,
          w4_ref, b4_ref, out_ref):
    def dot(a, b):
        return jax.lax.dot_general(a, b, (((1,), (0,)), ((), ())),
                                   preferred_element_type=jnp.float32)

    ci = ci_ref[...]
    k1 = _row_argmax_t(dot(jnp.tanh(dot(ci, wc11_ref[...]) + bc11_ref[...]),
                           wc12_ref[...]) + bc12_ref[...])
    k2 = _row_argmax_t(dot(jnp.tanh(dot(ci, wc21_ref[...]) + bc21_ref[...]),
                           wc22_ref[...]) + bc22_ref[...])
    k3 = _row_argmax_t(dot(jnp.tanh(dot(ci, wc31_ref[...]) + bc31_ref[...]),
                           wc32_ref[...]) + bc32_ref[...])
    x = _kwta(dot(ci, w1_ref[...]) + b1_ref[...], k1)
    x = _kwta(dot(x, w2_ref[...]) + b2_ref[...], k2)
    x = _kwta(dot(x, w3_ref[...]) + b3_ref[...], k3)
    out_ref[...] = dot(x, w4_ref[...]) + b4_ref[...]


def kernel(state, action, task_indicator,
           w_cx1_1, b_cx1_1, w_cx1_2, b_cx1_2,
           w_cx2_1, b_cx2_1, w_cx2_2, b_cx2_2,
           w_cx3_1, b_cx3_1, w_cx3_2, b_cx3_2,
           w1, b1, w2, b2, w3, b3, w4, b4):
    b = state.shape[0]
    rows = min(_ROWS, b)
    ci = jnp.concatenate([state, task_indicator, action], axis=1)
    inp = ci.shape[1]
    h = w4.shape[1]

    weights = [w_cx1_1, b_cx1_1.reshape(1, -1), w_cx1_2, b_cx1_2.reshape(1, -1),
               w_cx2_1, b_cx2_1.reshape(1, -1), w_cx2_2, b_cx2_2.reshape(1, -1),
               w_cx3_1, b_cx3_1.reshape(1, -1), w_cx3_2, b_cx3_2.reshape(1, -1),
               w1, b1.reshape(1, -1), w2, b2.reshape(1, -1),
               w3, b3.reshape(1, -1), w4, b4.reshape(1, -1)]

    return pl.pallas_call(
        _body,
        grid=(b // rows,),
        in_specs=[pl.BlockSpec((rows, inp), lambda i: (i, 0))] +
                 [pl.BlockSpec(w.shape, lambda i: (0, 0)) for w in weights],
        out_specs=pl.BlockSpec((rows, h), lambda i: (i, 0)),
        out_shape=jax.ShapeDtypeStruct((b, h), jnp.float32),
        compiler_params=pltpu.CompilerParams(
            dimension_semantics=("parallel",)),
    )(ci, *weights)
